# baseline (speedup 1.0000x reference)
import functools

import jax
import jax.numpy as jnp
from jax import lax
from jax.experimental import pallas as pl
from jax.experimental.pallas import tpu as pltpu
from jax.experimental.pallas import tpu_sc as plsc

_EMBEDDING_DIM = 1024

_INFO = plsc.get_sparse_core_info()
_NC, _NS = _INFO.num_cores, _INFO.num_subcores
_NW = _NC * _NS

_CH = 32
_NBUF = 3


def kernel(input, table):
    seq_len = input.shape[-1]
    rows_per_w = seq_len // _NW
    nchunks = rows_per_w // _CH
    pos_ids = jnp.arange(seq_len, dtype=jnp.int32)
    mesh = plsc.VectorSubcoreMesh(core_axis_name="c", subcore_axis_name="s")

    @functools.partial(
        pl.kernel,
        mesh=mesh,
        out_type=jax.ShapeDtypeStruct((1, seq_len, _EMBEDDING_DIM), table.dtype),
        scratch_types=[
            pltpu.VMEM((rows_per_w,), jnp.int32),
            pltpu.VMEM((_NBUF, _CH, _EMBEDDING_DIM), table.dtype),
            pltpu.SemaphoreType.DMA((_NBUF,)),
            pltpu.SemaphoreType.DMA((_NBUF,)),
        ],
    )
    def run(table_hbm, ids_hbm, out_hbm, idx_v, buf, in_sems, out_sems):
        wid = lax.axis_index("s") * _NC + lax.axis_index("c")
        base = wid * rows_per_w
        pltpu.sync_copy(ids_hbm.at[pl.ds(base, rows_per_w)], idx_v)

        def gather(c):
            return pltpu.make_async_copy(
                table_hbm.at[idx_v.at[pl.ds(c * _CH, _CH)]],
                buf.at[c % _NBUF],
                in_sems.at[c % _NBUF],
            )

        def out_copy(c):
            return pltpu.make_async_copy(
                buf.at[c % _NBUF],
                out_hbm.at[0].at[pl.ds(base + c * _CH, _CH)],
                out_sems.at[c % _NBUF],
            )

        for c in range(min(_NBUF, nchunks)):
            gather(c).start()
        for c in range(nchunks):
            gather(c).wait()
            out_copy(c).start()
            nxt = c + _NBUF
            if nxt < nchunks:
                out_copy(c).wait()
                gather(nxt).start()
        for c in range(max(nchunks - _NBUF, 0), nchunks):
            out_copy(c).wait()

    return run(table, pos_ids)


# speedup vs baseline: 1.0303x; 1.0303x over previous
import functools

import jax
import jax.numpy as jnp
from jax import lax
from jax.experimental import pallas as pl
from jax.experimental.pallas import tpu as pltpu
from jax.experimental.pallas import tpu_sc as plsc

_EMBEDDING_DIM = 1024

_INFO = plsc.get_sparse_core_info()
_NC, _NS = _INFO.num_cores, _INFO.num_subcores
_NW = _NC * _NS

_CH = 32
_NBUF = 3


def kernel(input, table):
    seq_len = input.shape[-1]
    rows_per_w = seq_len // _NW
    nchunks = rows_per_w // _CH
    mesh = plsc.VectorSubcoreMesh(core_axis_name="c", subcore_axis_name="s")

    @functools.partial(
        pl.kernel,
        mesh=mesh,
        out_type=jax.ShapeDtypeStruct((1, seq_len, _EMBEDDING_DIM), table.dtype),
        scratch_types=[
            pltpu.VMEM((_NBUF, _CH, _EMBEDDING_DIM), table.dtype),
            pltpu.SemaphoreType.DMA((_NBUF,)),
            pltpu.SemaphoreType.DMA((_NBUF,)),
        ],
    )
    def run(table_hbm, out_hbm, buf, in_sems, out_sems):
        wid = lax.axis_index("s") * _NC + lax.axis_index("c")
        base = wid * rows_per_w

        def in_copy(c):
            return pltpu.make_async_copy(
                table_hbm.at[pl.ds(base + c * _CH, _CH)],
                buf.at[c % _NBUF],
                in_sems.at[c % _NBUF],
            )

        def out_copy(c):
            return pltpu.make_async_copy(
                buf.at[c % _NBUF],
                out_hbm.at[0].at[pl.ds(base + c * _CH, _CH)],
                out_sems.at[c % _NBUF],
            )

        for c in range(min(_NBUF, nchunks)):
            in_copy(c).start()
        for c in range(nchunks):
            in_copy(c).wait()
            out_copy(c).start()
            nxt = c + _NBUF
            if nxt < nchunks:
                out_copy(c).wait()
                in_copy(nxt).start()
        for c in range(max(nchunks - _NBUF, 0), nchunks):
            out_copy(c).wait()

    return run(table)
